# baseline (speedup 1.0000x reference)
import functools

import jax
import jax.numpy as jnp
from jax import lax
from jax.experimental import pallas as pl
from jax.experimental.pallas import tpu as pltpu
from jax.experimental.pallas import tpu_sc as plsc

_B = 16384
_D = 64
_NW = 32
_BPW = _B // _NW
_CHUNK = 128
_NCH = _BPW // _CHUNK


def _sc_gather_body(uidx_hbm, iidx_hbm, uemb_hbm, iemb_hbm, u_out, v_out,
                    uidx_v, iidx_v, sem):
    wid = lax.axis_index("s") * 2 + lax.axis_index("c")
    base = wid * _BPW
    pltpu.sync_copy(uidx_hbm.at[pl.ds(base, _BPW)], uidx_v)
    pltpu.sync_copy(iidx_hbm.at[pl.ds(base, _BPW)], iidx_v)

    def chunk(c, carry):
        uvec = uidx_v[pl.ds(c * 16, 16)]
        ivec = iidx_v[pl.ds(c * 16, 16)]
        for l in range(16):
            k = base + c * 16 + l
            pltpu.async_copy(uemb_hbm.at[pl.ds(uvec[l], 1)],
                             u_out.at[pl.ds(k, 1)], sem)
            pltpu.async_copy(iemb_hbm.at[pl.ds(ivec[l], 1)],
                             v_out.at[pl.ds(k, 1)], sem)
        return carry

    lax.fori_loop(0, _BPW // 16, chunk, 0)
    pltpu.make_async_copy(uemb_hbm.at[pl.ds(0, _BPW)],
                          u_out.at[pl.ds(base, _BPW)], sem).wait()
    pltpu.make_async_copy(iemb_hbm.at[pl.ds(0, _BPW)],
                          v_out.at[pl.ds(base, _BPW)], sem).wait()


@jax.jit
def _sc_gather(user_idx, item_idx, user_emb, item_emb):
    mesh = plsc.VectorSubcoreMesh(core_axis_name="c", subcore_axis_name="s")
    f = pl.kernel(
        _sc_gather_body,
        out_type=(
            jax.ShapeDtypeStruct((_B, _D), jnp.float32),
            jax.ShapeDtypeStruct((_B, _D), jnp.float32),
        ),
        mesh=mesh,
        scratch_types=[
            pltpu.VMEM((_BPW,), jnp.int32),
            pltpu.VMEM((_BPW,), jnp.int32),
            pltpu.SemaphoreType.DMA,
        ],
    )
    return f(user_idx, item_idx, user_emb, item_emb)


_BLK = 1024


def _mlp_body(u_ref, v_ref, w1u_ref, w1v_ref, b1_ref, w2_ref, b2_ref,
              w3_ref, b3_ref, wo_ref, bo_ref, out_ref):
    h = u_ref[...] @ w1u_ref[...] + v_ref[...] @ w1v_ref[...] + b1_ref[...]
    h = jnp.maximum(h, 0.0)
    h = jnp.maximum(h @ w2_ref[...] + b2_ref[...], 0.0)
    h = jnp.maximum(h @ w3_ref[...] + b3_ref[...], 0.0)
    o = jnp.sum(h * wo_ref[...], axis=1, keepdims=True) + bo_ref[0, 0]
    out_ref[...] = jnp.maximum(o, 0.0)


@jax.jit
def _tc_mlp(u, v, W1, b1, W2, b2, W3, b3, Wo, bo):
    rep = lambda s: pl.BlockSpec(s, lambda i: (0,) * len(s))
    f = pl.pallas_call(
        _mlp_body,
        grid=(_B // _BLK,),
        in_specs=[
            pl.BlockSpec((_BLK, _D), lambda i: (i, 0)),
            pl.BlockSpec((_BLK, _D), lambda i: (i, 0)),
            rep((_D, 256)), rep((_D, 256)), rep((1, 256)),
            rep((256, 128)), rep((1, 128)),
            rep((128, 64)), rep((1, 64)),
            rep((1, 64)), rep((1, 1)),
        ],
        out_specs=pl.BlockSpec((_BLK, 1), lambda i: (i, 0)),
        out_shape=jax.ShapeDtypeStruct((_B, 1), jnp.float32),
    )
    return f(u, v, W1[:_D], W1[_D:], b1.reshape(1, -1), W2, b2.reshape(1, -1),
             W3, b3.reshape(1, -1), Wo.reshape(1, -1), bo.reshape(1, 1))


def kernel(user_idx, item_idx, user_emb, item_emb,
           W1, b1, W2, b2, W3, b3, Wo, bo):
    u, v = _sc_gather(user_idx, item_idx, user_emb, item_emb)
    return _tc_mlp(u, v, W1, b1, W2, b2, W3, b3, Wo, bo)


# speedup vs baseline: 1.6684x; 1.6684x over previous
import functools

import jax
import jax.numpy as jnp
from jax import lax
from jax.experimental import pallas as pl
from jax.experimental.pallas import tpu as pltpu
from jax.experimental.pallas import tpu_sc as plsc

_B = 16384
_D = 64
_NW = 32
_BPW = _B // _NW
_CHUNK = 128
_NCH = _BPW // _CHUNK


def _sc_gather_body(uidx_hbm, iidx_hbm, uemb_hbm, iemb_hbm, u_out, v_out,
                    uidx_v, iidx_v, rows_v, sem):
    wid = lax.axis_index("s") * 2 + lax.axis_index("c")
    base = wid * _BPW
    pltpu.sync_copy(uidx_hbm.at[pl.ds(base, _BPW)], uidx_v)
    pltpu.sync_copy(iidx_hbm.at[pl.ds(base, _BPW)], iidx_v)

    def one_table(idx_v, emb_hbm, out_hbm):
        def chunk(c, carry):
            vec = idx_v[pl.ds(c * 16, 16)]
            for l in range(16):
                pltpu.async_copy(emb_hbm.at[pl.ds(vec[l], 1)],
                                 rows_v.at[pl.ds(c * 16 + l, 1)], sem)
            return carry

        lax.fori_loop(0, _BPW // 16, chunk, 0)
        pltpu.make_async_copy(emb_hbm.at[pl.ds(0, _BPW)], rows_v, sem).wait()
        pltpu.sync_copy(rows_v, out_hbm.at[pl.ds(base, _BPW)])

    one_table(uidx_v, uemb_hbm, u_out)
    one_table(iidx_v, iemb_hbm, v_out)


@jax.jit
def _sc_gather(user_idx, item_idx, user_emb, item_emb):
    mesh = plsc.VectorSubcoreMesh(core_axis_name="c", subcore_axis_name="s")
    f = pl.kernel(
        _sc_gather_body,
        out_type=(
            jax.ShapeDtypeStruct((_B, _D), jnp.float32),
            jax.ShapeDtypeStruct((_B, _D), jnp.float32),
        ),
        mesh=mesh,
        scratch_types=[
            pltpu.VMEM((_BPW,), jnp.int32),
            pltpu.VMEM((_BPW,), jnp.int32),
            pltpu.VMEM((_BPW, _D), jnp.float32),
            pltpu.SemaphoreType.DMA,
        ],
    )
    return f(user_idx, item_idx, user_emb, item_emb)


_BLK = 1024


def _mlp_body(u_ref, v_ref, w1u_ref, w1v_ref, b1_ref, w2_ref, b2_ref,
              w3_ref, b3_ref, wo_ref, bo_ref, out_ref):
    h = u_ref[...] @ w1u_ref[...] + v_ref[...] @ w1v_ref[...] + b1_ref[...]
    h = jnp.maximum(h, 0.0)
    h = jnp.maximum(h @ w2_ref[...] + b2_ref[...], 0.0)
    h = jnp.maximum(h @ w3_ref[...] + b3_ref[...], 0.0)
    o = jnp.sum(h * wo_ref[...], axis=1, keepdims=True) + bo_ref[0, 0]
    out_ref[...] = jnp.maximum(o, 0.0)


@jax.jit
def _tc_mlp(u, v, W1, b1, W2, b2, W3, b3, Wo, bo):
    rep = lambda s: pl.BlockSpec(s, lambda i: (0,) * len(s))
    f = pl.pallas_call(
        _mlp_body,
        grid=(_B // _BLK,),
        in_specs=[
            pl.BlockSpec((_BLK, _D), lambda i: (i, 0)),
            pl.BlockSpec((_BLK, _D), lambda i: (i, 0)),
            rep((_D, 256)), rep((_D, 256)), rep((1, 256)),
            rep((256, 128)), rep((1, 128)),
            rep((128, 64)), rep((1, 64)),
            rep((1, 64)), rep((1, 1)),
        ],
        out_specs=pl.BlockSpec((_BLK, 1), lambda i: (i, 0)),
        out_shape=jax.ShapeDtypeStruct((_B, 1), jnp.float32),
    )
    return f(u, v, W1[:_D], W1[_D:], b1.reshape(1, -1), W2, b2.reshape(1, -1),
             W3, b3.reshape(1, -1), Wo.reshape(1, -1), bo.reshape(1, 1))


def kernel(user_idx, item_idx, user_emb, item_emb,
           W1, b1, W2, b2, W3, b3, Wo, bo):
    u, v = _sc_gather(user_idx, item_idx, user_emb, item_emb)
    return _tc_mlp(u, v, W1, b1, W2, b2, W3, b3, Wo, bo)


# speedup vs baseline: 1.6783x; 1.0059x over previous
import functools

import jax
import jax.numpy as jnp
from jax import lax
from jax.experimental import pallas as pl
from jax.experimental.pallas import tpu as pltpu
from jax.experimental.pallas import tpu_sc as plsc

_B = 16384
_D = 64
_NW = 32
_BPW = _B // _NW


def _sc_gather_body(idx_hbm, emb_hbm, out_hbm, idx_v, rows_v, sem):
    wid = lax.axis_index("s") * 2 + lax.axis_index("c")
    base = wid * _BPW
    pltpu.sync_copy(idx_hbm.at[pl.ds(base, _BPW)], idx_v)

    def chunk(c, carry):
        vec = idx_v[pl.ds(c * 16, 16)]
        for l in range(16):
            pltpu.async_copy(emb_hbm.at[pl.ds(vec[l], 1)],
                             rows_v.at[pl.ds(c * 16 + l, 1)], sem)
        return carry

    lax.fori_loop(0, _BPW // 16, chunk, 0)
    pltpu.make_async_copy(emb_hbm.at[pl.ds(0, _BPW)], rows_v, sem).wait()
    pltpu.sync_copy(rows_v, out_hbm.at[pl.ds(base, _BPW)])


@jax.jit
def _sc_gather(idx, emb):
    mesh = plsc.VectorSubcoreMesh(core_axis_name="c", subcore_axis_name="s")
    f = pl.kernel(
        _sc_gather_body,
        out_type=jax.ShapeDtypeStruct((_B, _D), jnp.float32),
        mesh=mesh,
        scratch_types=[
            pltpu.VMEM((_BPW,), jnp.int32),
            pltpu.VMEM((_BPW, _D), jnp.float32),
            pltpu.SemaphoreType.DMA,
        ],
    )
    return f(idx, emb)


_BLK = 1024


def _mlp_body(u_ref, v_ref, w1u_ref, w1v_ref, b1_ref, w2_ref, b2_ref,
              w3_ref, b3_ref, wo_ref, bo_ref, out_ref):
    h = u_ref[...] @ w1u_ref[...] + v_ref[...] @ w1v_ref[...] + b1_ref[...]
    h = jnp.maximum(h, 0.0)
    h = jnp.maximum(h @ w2_ref[...] + b2_ref[...], 0.0)
    h = jnp.maximum(h @ w3_ref[...] + b3_ref[...], 0.0)
    o = jnp.sum(h * wo_ref[...], axis=1, keepdims=True) + bo_ref[0, 0]
    out_ref[...] = jnp.maximum(o, 0.0)


@jax.jit
def _tc_mlp(u, v, W1, b1, W2, b2, W3, b3, Wo, bo):
    rep = lambda s: pl.BlockSpec(s, lambda i: (0,) * len(s))
    f = pl.pallas_call(
        _mlp_body,
        grid=(_B // _BLK,),
        in_specs=[
            pl.BlockSpec((_BLK, _D), lambda i: (i, 0)),
            pl.BlockSpec((_BLK, _D), lambda i: (i, 0)),
            rep((_D, 256)), rep((_D, 256)), rep((1, 256)),
            rep((256, 128)), rep((1, 128)),
            rep((128, 64)), rep((1, 64)),
            rep((1, 64)), rep((1, 1)),
        ],
        out_specs=pl.BlockSpec((_BLK, 1), lambda i: (i, 0)),
        out_shape=jax.ShapeDtypeStruct((_B, 1), jnp.float32),
    )
    return f(u, v, W1[:_D], W1[_D:], b1.reshape(1, -1), W2, b2.reshape(1, -1),
             W3, b3.reshape(1, -1), Wo.reshape(1, -1), bo.reshape(1, 1))


def kernel(user_idx, item_idx, user_emb, item_emb,
           W1, b1, W2, b2, W3, b3, Wo, bo):
    u = _sc_gather(user_idx, user_emb)
    v = _sc_gather(item_idx, item_emb)
    return _tc_mlp(u, v, W1, b1, W2, b2, W3, b3, Wo, bo)
